# baseline (speedup 1.0000x reference)
import functools

import jax
import jax.numpy as jnp
from jax.experimental import pallas as pl
from jax.experimental.pallas import tpu as pltpu
from jax.experimental.pallas import tpu_sc as plsc

_N = 4096
_K = 32768
_C = 4
_KT = 512
_NT = _K // _KT
_BETA = 0.25

_NC = 2
_NS = 16
_NW = _NC * _NS
_BPW = _N // _NW
_DP = 16


def _argmin_body(z_ref, cbtm2_ref, idx_ref):
    z = z_ref[...]
    zsq = jnp.sum(z * z, axis=1, keepdims=True)

    def tile_d(t):
        cm2 = cbtm2_ref[:, pl.ds(t * _KT, _KT)]
        csq = 0.25 * jnp.sum(cm2 * cm2, axis=0, keepdims=True)
        m2n = jax.lax.dot_general(
            z, cm2, (((1,), (0,)), ((), ())),
            preferred_element_type=jnp.float32)
        return (zsq + csq) + m2n

    best_d = tile_d(0)
    best_t = jnp.zeros((_N, _KT), jnp.int32)
    for t in range(1, _NT):
        d = tile_d(t)
        upd = d < best_d
        best_d = jnp.where(upd, d, best_d)
        best_t = jnp.where(upd, t, best_t)

    dmin = jnp.min(best_d, axis=1, keepdims=True)
    lane = jax.lax.broadcasted_iota(jnp.int32, (_N, _KT), 1)
    gidx = best_t * _KT + lane
    sel = jnp.where(best_d == dmin, gidx, jnp.int32(_K))
    idx_ref[...] = jnp.min(sel, axis=1, keepdims=True)


def _argmin_call(z_flat, cbt_m2):
    return pl.pallas_call(
        _argmin_body,
        out_shape=jax.ShapeDtypeStruct((_N, 1), jnp.int32),
    )(z_flat, cbt_m2)


def _sc_gather_body(cb_hbm, idx_hbm, z_hbm, zq_hbm, part_hbm,
                    idx_v, rows_v, z_v, out_v, acc_v, sem):
    wid = jax.lax.axis_index("s") * _NC + jax.lax.axis_index("c")
    base = wid * _BPW
    pltpu.sync_copy(idx_hbm.at[pl.ds(base, _BPW)], idx_v)
    pltpu.async_copy(cb_hbm.at[idx_v], rows_v, sem).wait()
    pltpu.sync_copy(z_hbm.at[pl.ds(base, _BPW)], z_v)

    def row(i, acc):
        e = rows_v[i, pl.ds(0, _DP)] - z_v[i]
        out_v[i] = z_v[i] + e
        return acc + e * e

    acc = jax.lax.fori_loop(0, _BPW, row, jnp.zeros((_DP,), jnp.float32))
    acc_v[0] = acc
    pltpu.sync_copy(out_v, zq_hbm.at[pl.ds(base, _BPW)])
    pltpu.sync_copy(acc_v, part_hbm.at[pl.ds(wid, 1)])


@functools.partial(jax.jit, static_argnames=())
def _sc_gather_call(cb_pad, idx_flat, z_pad):
    mesh = plsc.VectorSubcoreMesh(core_axis_name="c", subcore_axis_name="s")
    f = pl.kernel(
        _sc_gather_body,
        mesh=mesh,
        out_type=(
            jax.ShapeDtypeStruct((_N, _DP), jnp.float32),
            jax.ShapeDtypeStruct((_NW, _DP), jnp.float32),
        ),
        scratch_types=[
            pltpu.VMEM((_BPW,), jnp.int32),
            pltpu.VMEM((_BPW, 128), jnp.float32),
            pltpu.VMEM((_BPW, _DP), jnp.float32),
            pltpu.VMEM((_BPW, _DP), jnp.float32),
            pltpu.VMEM((1, _DP), jnp.float32),
            pltpu.SemaphoreType.DMA,
        ],
    )
    return f(cb_pad, idx_flat, z_pad)


def kernel(z, codebook):
    b, c, h, w = z.shape
    z_flat = jnp.transpose(z, (0, 2, 3, 1)).reshape(-1, c)
    cbt_m2 = codebook.T * jnp.float32(-2.0)
    idx = _argmin_call(z_flat, cbt_m2)

    cb_pad = jnp.pad(codebook, ((0, 0), (0, 128 - _C)))
    z_pad = jnp.pad(z_flat, ((0, 0), (0, _DP - _C)))
    zq_pad, parts = _sc_gather_call(cb_pad, idx.reshape(_N), z_pad)

    m = jnp.sum(parts) * jnp.float32(1.0 / (_N * _C))
    loss = _BETA * m + m
    zq_st = zq_pad[:, :_C]
    z_q_out = jnp.transpose(zq_st.reshape(b, h, w, c), (0, 3, 1, 2))
    indices = idx.reshape(b, h, w)
    return z_q_out, loss, indices
